# Initial kernel scaffold; baseline (speedup 1.0000x reference)
import jax
import jax.numpy as jnp
from jax.experimental import pallas as pl


def kernel(x_scalar, x_vector, rbf, envelope, rsh, edge_index, W1, b1, W2, b2, Wr, br):
    raise NotImplementedError("write your pallas kernel here")


# speedup vs baseline: 9.3342x; 9.3342x over previous
import jax
import jax.numpy as jnp
from jax import lax
from jax.experimental import pallas as pl
from jax.experimental.pallas import tpu as pltpu
from jax.experimental.pallas import tpu_sc as plsc

N, E = 10000, 320000
ND, ED, NB = 128, 128, 20
HID = ND + 2 * ED
PW = 512
NPAD = 10240
RANGE = 1280
ROUNDS = 4
ACC_ROWS = RANGE + 16
DUMMY = RANGE
NC, NS, L = 2, 16, 16
EPT = E // NS
SEG = 2000
NSEG = EPT // SEG
CCAP = SEG + 16
TRASH = SEG
K = 32
ROWS_PT = RANGE // NS


def _mlp_body(x_ref, w1_ref, b1_ref, w2_ref, b2_ref, o_ref):
    h = jnp.dot(x_ref[...], w1_ref[...], preferred_element_type=jnp.float32)
    h = h + b1_ref[...]
    h = h * jax.nn.sigmoid(h)
    o_ref[...] = jnp.dot(h, w2_ref[...], preferred_element_type=jnp.float32) + b2_ref[...]


def _mlp(x, W1, b1, W2, b2):
    blk = 512
    return pl.pallas_call(
        _mlp_body,
        grid=(NPAD // blk,),
        in_specs=[
            pl.BlockSpec((blk, ND), lambda i: (i, 0)),
            pl.BlockSpec((ND, ND), lambda i: (0, 0)),
            pl.BlockSpec((1, ND), lambda i: (0, 0)),
            pl.BlockSpec((ND, HID), lambda i: (0, 0)),
            pl.BlockSpec((1, HID), lambda i: (0, 0)),
        ],
        out_specs=pl.BlockSpec((blk, HID), lambda i: (i, 0)),
        out_shape=jax.ShapeDtypeStruct((NPAD, HID), jnp.float32),
    )(x, W1, b1.reshape(1, ND), W2, b2.reshape(1, HID))


def _edge_pack_body(rbf_ref, env_ref, rsh_ref, wr_ref, br_ref, o_ref):
    t = jnp.dot(rbf_ref[...], wr_ref[...], preferred_element_type=jnp.float32)
    fw = (t + br_ref[...]) * env_ref[...]
    rshp = jnp.pad(rsh_ref[...], ((0, 0), (0, PW - HID - 4)))
    o_ref[...] = jnp.concatenate([fw, rshp], axis=1)


def _edge_pack(rbf, envelope, rsh4, Wr, br):
    blk = 1280
    return pl.pallas_call(
        _edge_pack_body,
        grid=(E // blk,),
        in_specs=[
            pl.BlockSpec((blk, NB), lambda i: (i, 0)),
            pl.BlockSpec((blk, 1), lambda i: (i, 0)),
            pl.BlockSpec((blk, 4), lambda i: (i, 0)),
            pl.BlockSpec((NB, HID), lambda i: (0, 0)),
            pl.BlockSpec((1, HID), lambda i: (0, 0)),
        ],
        out_specs=pl.BlockSpec((blk, PW), lambda i: (i, 0)),
        out_shape=jax.ShapeDtypeStruct((E, PW), jnp.float32),
    )(rbf, envelope, rsh4, Wr, br.reshape(1, HID))


def _sc_body(sout_hbm, xvec_hbm, p_hbm, src_hbm, dst_hbm,
             x0_hbm, x1_hbm, x2_hbm, x3_hbm,
             o0_hbm, o1_hbm, o2_hbm, o3_hbm,
             acc0, acc1, acc2, acc3, ebuf, sbuf, dbuf,
             dseg, sseg, posb, dst_st, ep, dummy, ecomp, scomp,
             didx, gS, gV, gP, m0, m1, m2, m3, sem):
    c = lax.axis_index("c")
    s = lax.axis_index("s")
    lanes = lax.iota(jnp.int32, L)
    sh_idx = [jnp.maximum(lanes - sh, 0) for sh in (1, 2, 4, 8)]
    sh_msk = [lanes >= sh for sh in (1, 2, 4, 8)]
    region = s * CCAP
    accs = (acc0, acc1, acc2, acc3)
    xs = (x0_hbm, x1_hbm, x2_hbm, x3_hbm)
    os_ = (o0_hbm, o1_hbm, o2_hbm, o3_hbm)
    msgs = (m0, m1, m2, m3)

    def fill(i, _):
        dummy[pl.ds(i * L, L)] = jnp.zeros((L,), jnp.int32) + DUMMY
        return 0
    lax.fori_loop(0, CCAP // L, fill, 0)

    def fill2(i, _):
        ep[pl.ds(i * L, L)] = i * L + lanes
        return 0
    lax.fori_loop(0, SEG // L, fill2, 0)

    def per_round(r, _):
        base = (2 * r + c) * RANGE

        row0 = s * ROWS_PT
        for q in range(4):
            pltpu.sync_copy(xs[q].at[pl.ds(base + row0, ROWS_PT)],
                            accs[q].at[pl.ds(row0, ROWS_PT)])
        @pl.when(s == 0)
        def _():
            for q in range(4):
                pltpu.sync_copy(xs[q].at[pl.ds(0, 16)],
                                accs[q].at[pl.ds(RANGE, 16)])
        plsc.subcore_barrier()

        def per_seg(g, _):
            row = s * NSEG + g
            e0 = row * SEG
            pltpu.sync_copy(dst_hbm.at[row], dseg)
            pltpu.sync_copy(src_hbm.at[row], sseg)

            def scan(i, cnt):
                d = dseg[pl.ds(i * L, L)]
                dl = d - base
                m = (dl >= 0) & (dl < RANGE)
                v = jnp.where(m, 1, 0)
                for ix, mk in zip(sh_idx, sh_msk):
                    g2 = v.at[ix].get(mode="promise_in_bounds")
                    v = v + jnp.where(mk, g2, 0)
                posb[pl.ds(i * L, L)] = jnp.where(m, cnt + v - 1, TRASH) + region
                dst_st[pl.ds(i * L, L)] = jnp.where(m, dl, DUMMY)
                return cnt + v[L - 1]

            cnt = lax.fori_loop(0, SEG // L, scan, jnp.int32(0))

            pltpu.sync_copy(dummy, dbuf.at[pl.ds(region, CCAP)])
            pltpu.sync_copy(ep, ebuf.at[posb])
            pltpu.sync_copy(sseg, sbuf.at[posb])
            pltpu.sync_copy(dst_st, dbuf.at[posb])

            pltpu.sync_copy(ebuf.at[pl.ds(region, CCAP)], ecomp)
            pltpu.sync_copy(sbuf.at[pl.ds(region, CCAP)], scomp)

            def shift(i, _):
                ev = ecomp[pl.ds(i * L, L)]
                ecomp[pl.ds(i * L, L)] = (
                    jnp.minimum(jnp.maximum(ev, 0), SEG - 1) + e0)
                sv = scomp[pl.ds(i * L, L)]
                scomp[pl.ds(i * L, L)] = jnp.minimum(jnp.maximum(sv, 0), N - 1)
                return 0
            lax.fori_loop(0, CCAP // L, shift, 0)

            nchunks = (cnt + K - 1) // K

            def chunk(j, _):
                off = j * K
                pltpu.sync_copy(dbuf.at[pl.ds(region + off, K)], didx)
                c1 = pltpu.async_copy(sout_hbm.at[scomp.at[pl.ds(off, K)]], gS, sem)
                c2 = pltpu.async_copy(xvec_hbm.at[scomp.at[pl.ds(off, K)]], gV, sem)
                c3 = pltpu.async_copy(p_hbm.at[ecomp.at[pl.ds(off, K)]], gP, sem)
                c1.wait(); c2.wait(); c3.wait()

                def edge(k, _):
                    rv = gP[k, pl.ds(HID, L)]
                    for grp in range(ND // L):
                        lo = grp * L
                        m0[k, pl.ds(lo, L)] = gS[k, pl.ds(lo, L)] * gP[k, pl.ds(lo, L)]
                        gev = gS[k, pl.ds(ND + lo, L)] * gP[k, pl.ds(ND + lo, L)]
                        gsv = gS[k, pl.ds(2 * ND + lo, L)] * gP[k, pl.ds(2 * ND + lo, L)]
                        m1[k, pl.ds(lo, L)] = gV[k, pl.ds(lo, L)] * gsv + gev * rv[0]
                        m2[k, pl.ds(lo, L)] = gV[k, pl.ds(ED + lo, L)] * gsv + gev * rv[1]
                        m3[k, pl.ds(lo, L)] = gV[k, pl.ds(2 * ED + lo, L)] * gsv + gev * rv[2]
                    return 0

                lax.fori_loop(0, K, edge, 0)
                for q in range(4):
                    pltpu.sync_copy(msgs[q], accs[q].at[didx], add=True)
                return 0

            lax.fori_loop(0, nchunks, chunk, 0)
            return 0

        lax.fori_loop(0, NSEG, per_seg, 0)
        plsc.subcore_barrier()

        for q in range(4):
            pltpu.sync_copy(accs[q].at[pl.ds(row0, ROWS_PT)],
                            os_[q].at[pl.ds(base + row0, ROWS_PT)])
        plsc.subcore_barrier()
        return 0

    lax.fori_loop(0, ROUNDS, per_round, 0)


def _sc_message(sout, xvec, p, src, dst, xq):
    mesh = plsc.VectorSubcoreMesh(core_axis_name="c", subcore_axis_name="s",
                                  num_cores=NC, num_subcores=NS)
    f32, i32 = jnp.float32, jnp.int32
    out_t = jax.ShapeDtypeStruct((NPAD, ND), f32)
    kfn = pl.kernel(
        _sc_body,
        out_type=(out_t, out_t, out_t, out_t),
        mesh=mesh,
        scratch_types=[
            pltpu.VMEM_SHARED((ACC_ROWS, ND), f32),
            pltpu.VMEM_SHARED((ACC_ROWS, ND), f32),
            pltpu.VMEM_SHARED((ACC_ROWS, ND), f32),
            pltpu.VMEM_SHARED((ACC_ROWS, ND), f32),
            pltpu.VMEM_SHARED((NS * CCAP,), i32),
            pltpu.VMEM_SHARED((NS * CCAP,), i32),
            pltpu.VMEM_SHARED((NS * CCAP,), i32),
            pltpu.VMEM((SEG,), i32),
            pltpu.VMEM((SEG,), i32),
            pltpu.VMEM((SEG,), i32),
            pltpu.VMEM((SEG,), i32),
            pltpu.VMEM((SEG,), i32),
            pltpu.VMEM((CCAP,), i32),
            pltpu.VMEM((CCAP,), i32),
            pltpu.VMEM((CCAP,), i32),
            pltpu.VMEM((K,), i32),
            pltpu.VMEM((K, HID), f32),
            pltpu.VMEM((K, 3 * ED), f32),
            pltpu.VMEM((K, PW), f32),
            pltpu.VMEM((K, ND), f32),
            pltpu.VMEM((K, ND), f32),
            pltpu.VMEM((K, ND), f32),
            pltpu.VMEM((K, ND), f32),
            pltpu.SemaphoreType.DMA,
        ],
    )
    return kfn(sout, xvec, p, src, dst, *xq)


def kernel(x_scalar, x_vector, rbf, envelope, rsh, edge_index, W1, b1, W2, b2, Wr, br):
    xs_pad = jnp.pad(x_scalar, ((0, NPAD - N), (0, 0)))
    sout = _mlp(xs_pad, W1, b1, W2, b2)
    rsh4 = jnp.pad(rsh, ((0, 0), (0, 1)))
    p = _edge_pack(rbf, envelope, rsh4, Wr, br)

    xvec = x_vector.reshape(N, 3 * ED)
    src = edge_index[1].astype(jnp.int32).reshape(E // SEG, SEG)
    dst = edge_index[0].astype(jnp.int32).reshape(E // SEG, SEG)
    pad_n = ((0, NPAD - N), (0, 0))
    xq = [xs_pad] + [jnp.pad(x_vector[:, q, :], pad_n) for q in range(3)]

    o0, o1, o2, o3 = _sc_message(sout, xvec, p, src, dst, xq)
    new_scalar = o0[:N]
    new_vector = jnp.stack([o1[:N], o2[:N], o3[:N]], axis=1)
    return new_scalar, new_vector


# speedup vs baseline: 11.7916x; 1.2633x over previous
import jax
import jax.numpy as jnp
from jax import lax
from jax.experimental import pallas as pl
from jax.experimental.pallas import tpu as pltpu
from jax.experimental.pallas import tpu_sc as plsc

N, E = 10000, 320000
ND, ED, NB = 128, 128, 20
HID = ND + 2 * ED
PW = 512
NPAD = 10240
RANGE = 1280
ROUNDS = 4
ACC_ROWS = RANGE + 16
DUMMY = RANGE
NC, NS, L = 2, 16, 16
EPT = E // NS
SEG = 2000
NSEG = EPT // SEG
CCAP = SEG + 16
TRASH = SEG
K = 16
ROWS_PT = RANGE // NS


def _mlp_body(x_ref, w1_ref, b1_ref, w2_ref, b2_ref, o_ref):
    h = jnp.dot(x_ref[...], w1_ref[...], preferred_element_type=jnp.float32)
    h = h + b1_ref[...]
    h = h * jax.nn.sigmoid(h)
    o_ref[...] = jnp.dot(h, w2_ref[...], preferred_element_type=jnp.float32) + b2_ref[...]


def _mlp(x, W1, b1, W2, b2):
    blk = 512
    return pl.pallas_call(
        _mlp_body,
        grid=(NPAD // blk,),
        in_specs=[
            pl.BlockSpec((blk, ND), lambda i: (i, 0)),
            pl.BlockSpec((ND, ND), lambda i: (0, 0)),
            pl.BlockSpec((1, ND), lambda i: (0, 0)),
            pl.BlockSpec((ND, HID), lambda i: (0, 0)),
            pl.BlockSpec((1, HID), lambda i: (0, 0)),
        ],
        out_specs=pl.BlockSpec((blk, HID), lambda i: (i, 0)),
        out_shape=jax.ShapeDtypeStruct((NPAD, HID), jnp.float32),
    )(x, W1, b1.reshape(1, ND), W2, b2.reshape(1, HID))


def _edge_pack_body(rbf_ref, env_ref, rsh_ref, wr_ref, br_ref, o_ref):
    t = jnp.dot(rbf_ref[...], wr_ref[...], preferred_element_type=jnp.float32)
    fw = (t + br_ref[...]) * env_ref[...]
    rshp = jnp.pad(rsh_ref[...], ((0, 0), (0, PW - HID - 4)))
    o_ref[...] = jnp.concatenate([fw, rshp], axis=1)


def _edge_pack(rbf, envelope, rsh4, Wr, br):
    blk = 1280
    return pl.pallas_call(
        _edge_pack_body,
        grid=(E // blk,),
        in_specs=[
            pl.BlockSpec((blk, NB), lambda i: (i, 0)),
            pl.BlockSpec((blk, 1), lambda i: (i, 0)),
            pl.BlockSpec((blk, 4), lambda i: (i, 0)),
            pl.BlockSpec((NB, HID), lambda i: (0, 0)),
            pl.BlockSpec((1, HID), lambda i: (0, 0)),
        ],
        out_specs=pl.BlockSpec((blk, PW), lambda i: (i, 0)),
        out_shape=jax.ShapeDtypeStruct((E, PW), jnp.float32),
    )(rbf, envelope, rsh4, Wr, br.reshape(1, HID))


def _sc_body(sout_hbm, xvec_hbm, p_hbm, src_hbm, dst_hbm,
             x0_hbm, x1_hbm, x2_hbm, x3_hbm,
             o0_hbm, o1_hbm, o2_hbm, o3_hbm,
             acc0, acc1, acc2, acc3, ebuf, sbuf, dbuf,
             dseg, sseg, posb, dst_st, ep, dummy, ecomp, scomp,
             didxA, didxB, gSA, gSB, gVA, gVB, gPA, gPB,
             mA0, mA1, mA2, mA3, mB0, mB1, mB2, mB3,
             gsemA, gsemB, ssemA, ssemB):
    c = lax.axis_index("c")
    s = lax.axis_index("s")
    lanes = lax.iota(jnp.int32, L)
    sh_idx = [jnp.maximum(lanes - sh, 0) for sh in (1, 2, 4, 8)]
    sh_msk = [lanes >= sh for sh in (1, 2, 4, 8)]
    region = s * CCAP
    accs = (acc0, acc1, acc2, acc3)
    xs = (x0_hbm, x1_hbm, x2_hbm, x3_hbm)
    os_ = (o0_hbm, o1_hbm, o2_hbm, o3_hbm)
    gS = (gSA, gSB)
    gV = (gVA, gVB)
    gP = (gPA, gPB)
    didx = (didxA, didxB)
    msgs = ((mA0, mA1, mA2, mA3), (mB0, mB1, mB2, mB3))
    gsems = (gsemA, gsemB)
    ssems = (ssemA, ssemB)

    def fill(i, _):
        dummy[pl.ds(i * L, L)] = jnp.zeros((L,), jnp.int32) + DUMMY
        return 0
    lax.fori_loop(0, CCAP // L, fill, 0)

    def fill2(i, _):
        ep[pl.ds(i * L, L)] = i * L + lanes
        return 0
    lax.fori_loop(0, SEG // L, fill2, 0)

    def per_round(r, _):
        base = (2 * r + c) * RANGE

        row0 = s * ROWS_PT
        for q in range(4):
            pltpu.sync_copy(xs[q].at[pl.ds(base + row0, ROWS_PT)],
                            accs[q].at[pl.ds(row0, ROWS_PT)])
        @pl.when(s == 0)
        def _():
            for q in range(4):
                pltpu.sync_copy(xs[q].at[pl.ds(0, 16)],
                                accs[q].at[pl.ds(RANGE, 16)])
        plsc.subcore_barrier()

        def per_seg(g, _):
            row = s * NSEG + g
            e0 = row * SEG
            pltpu.sync_copy(dst_hbm.at[row], dseg)
            pltpu.sync_copy(src_hbm.at[row], sseg)

            def scan(i, cnt):
                d = dseg[pl.ds(i * L, L)]
                dl = d - base
                m = (dl >= 0) & (dl < RANGE)
                v = jnp.where(m, 1, 0)
                for ix, mk in zip(sh_idx, sh_msk):
                    g2 = v.at[ix].get(mode="promise_in_bounds")
                    v = v + jnp.where(mk, g2, 0)
                posb[pl.ds(i * L, L)] = jnp.where(m, cnt + v - 1, TRASH) + region
                dst_st[pl.ds(i * L, L)] = jnp.where(m, dl, DUMMY)
                return cnt + v[L - 1]

            cnt = lax.fori_loop(0, SEG // L, scan, jnp.int32(0))

            pltpu.sync_copy(dummy, dbuf.at[pl.ds(region, CCAP)])
            pltpu.sync_copy(ep, ebuf.at[posb])
            pltpu.sync_copy(sseg, sbuf.at[posb])
            pltpu.sync_copy(dst_st, dbuf.at[posb])

            pltpu.sync_copy(ebuf.at[pl.ds(region, CCAP)], ecomp)
            pltpu.sync_copy(sbuf.at[pl.ds(region, CCAP)], scomp)

            def shift(i, _):
                ev = ecomp[pl.ds(i * L, L)]
                ecomp[pl.ds(i * L, L)] = (
                    jnp.minimum(jnp.maximum(ev, 0), SEG - 1) + e0)
                sv = scomp[pl.ds(i * L, L)]
                scomp[pl.ds(i * L, L)] = jnp.minimum(jnp.maximum(sv, 0), N - 1)
                return 0
            lax.fori_loop(0, CCAP // L, shift, 0)

            nchunks = (cnt + K - 1) // K

            def step(j, _):
                for p in (0, 1):
                    q = 1 - p

                    @pl.when(j % 2 == p)
                    def _():
                        @pl.when(j < nchunks)
                        def _():
                            off = j * K
                            pltpu.async_copy(
                                sout_hbm.at[scomp.at[pl.ds(off, K)]], gS[p], gsems[p])
                            pltpu.async_copy(
                                xvec_hbm.at[scomp.at[pl.ds(off, K)]], gV[p], gsems[p])
                            pltpu.async_copy(
                                p_hbm.at[ecomp.at[pl.ds(off, K)]], gP[p], gsems[p])

                        @pl.when(j >= 1)
                        def _():
                            pltpu.make_async_copy(
                                sout_hbm.at[pl.ds(0, K)], gS[q], gsems[q]).wait()
                            pltpu.make_async_copy(
                                xvec_hbm.at[pl.ds(0, K)], gV[q], gsems[q]).wait()
                            pltpu.make_async_copy(
                                p_hbm.at[pl.ds(0, K)], gP[q], gsems[q]).wait()

                            @pl.when(j >= 3)
                            def _():
                                for i in range(4):
                                    pltpu.make_async_copy(
                                        x0_hbm.at[pl.ds(0, K)], msgs[q][i],
                                        ssems[q]).wait()

                            off1 = (j - 1) * K
                            pltpu.sync_copy(dbuf.at[pl.ds(region + off1, K)],
                                            didx[q])

                            def edge(k, _):
                                rv = gP[q][k, pl.ds(HID, L)]
                                for grp in range(ND // L):
                                    lo = grp * L
                                    msgs[q][0][k, pl.ds(lo, L)] = (
                                        gS[q][k, pl.ds(lo, L)] * gP[q][k, pl.ds(lo, L)])
                                    gev = gS[q][k, pl.ds(ND + lo, L)] * gP[q][k, pl.ds(ND + lo, L)]
                                    gsv = gS[q][k, pl.ds(2 * ND + lo, L)] * gP[q][k, pl.ds(2 * ND + lo, L)]
                                    msgs[q][1][k, pl.ds(lo, L)] = (
                                        gV[q][k, pl.ds(lo, L)] * gsv + gev * rv[0])
                                    msgs[q][2][k, pl.ds(lo, L)] = (
                                        gV[q][k, pl.ds(ED + lo, L)] * gsv + gev * rv[1])
                                    msgs[q][3][k, pl.ds(lo, L)] = (
                                        gV[q][k, pl.ds(2 * ED + lo, L)] * gsv + gev * rv[2])
                                return 0

                            lax.fori_loop(0, K, edge, 0)
                            for i in range(4):
                                pltpu.async_copy(msgs[q][i], accs[i].at[didx[q]],
                                                 ssems[q], add=True)
                return 0

            lax.fori_loop(0, nchunks + 1, step, 0)

            for p in (0, 1):
                @pl.when(nchunks >= 1 + p)
                def _():
                    par = (nchunks - 1 - p) % 2
                    for par2 in (0, 1):
                        @pl.when(par == par2)
                        def _():
                            for i in range(4):
                                pltpu.make_async_copy(
                                    x0_hbm.at[pl.ds(0, K)], msgs[par2][i],
                                    ssems[par2]).wait()
            return 0

        lax.fori_loop(0, NSEG, per_seg, 0)
        plsc.subcore_barrier()

        for q in range(4):
            pltpu.sync_copy(accs[q].at[pl.ds(row0, ROWS_PT)],
                            os_[q].at[pl.ds(base + row0, ROWS_PT)])
        plsc.subcore_barrier()
        return 0

    lax.fori_loop(0, ROUNDS, per_round, 0)


def _sc_message(sout, xvec, p, src, dst, xq):
    mesh = plsc.VectorSubcoreMesh(core_axis_name="c", subcore_axis_name="s",
                                  num_cores=NC, num_subcores=NS)
    f32, i32 = jnp.float32, jnp.int32
    out_t = jax.ShapeDtypeStruct((NPAD, ND), f32)
    kfn = pl.kernel(
        _sc_body,
        out_type=(out_t, out_t, out_t, out_t),
        mesh=mesh,
        scratch_types=[
            pltpu.VMEM_SHARED((ACC_ROWS, ND), f32),
            pltpu.VMEM_SHARED((ACC_ROWS, ND), f32),
            pltpu.VMEM_SHARED((ACC_ROWS, ND), f32),
            pltpu.VMEM_SHARED((ACC_ROWS, ND), f32),
            pltpu.VMEM_SHARED((NS * CCAP,), i32),
            pltpu.VMEM_SHARED((NS * CCAP,), i32),
            pltpu.VMEM_SHARED((NS * CCAP,), i32),
            pltpu.VMEM((SEG,), i32),
            pltpu.VMEM((SEG,), i32),
            pltpu.VMEM((SEG,), i32),
            pltpu.VMEM((SEG,), i32),
            pltpu.VMEM((SEG,), i32),
            pltpu.VMEM((CCAP,), i32),
            pltpu.VMEM((CCAP,), i32),
            pltpu.VMEM((CCAP,), i32),
            pltpu.VMEM((K,), i32),
            pltpu.VMEM((K,), i32),
            pltpu.VMEM((K, HID), f32),
            pltpu.VMEM((K, HID), f32),
            pltpu.VMEM((K, 3 * ED), f32),
            pltpu.VMEM((K, 3 * ED), f32),
            pltpu.VMEM((K, PW), f32),
            pltpu.VMEM((K, PW), f32),
            pltpu.VMEM((K, ND), f32),
            pltpu.VMEM((K, ND), f32),
            pltpu.VMEM((K, ND), f32),
            pltpu.VMEM((K, ND), f32),
            pltpu.VMEM((K, ND), f32),
            pltpu.VMEM((K, ND), f32),
            pltpu.VMEM((K, ND), f32),
            pltpu.VMEM((K, ND), f32),
            pltpu.SemaphoreType.DMA,
            pltpu.SemaphoreType.DMA,
            pltpu.SemaphoreType.DMA,
            pltpu.SemaphoreType.DMA,
        ],
    )
    return kfn(sout, xvec, p, src, dst, *xq)


def kernel(x_scalar, x_vector, rbf, envelope, rsh, edge_index, W1, b1, W2, b2, Wr, br):
    xs_pad = jnp.pad(x_scalar, ((0, NPAD - N), (0, 0)))
    sout = _mlp(xs_pad, W1, b1, W2, b2)
    rsh4 = jnp.pad(rsh, ((0, 0), (0, 1)))
    p = _edge_pack(rbf, envelope, rsh4, Wr, br)

    xvec = x_vector.reshape(N, 3 * ED)
    src = edge_index[1].astype(jnp.int32).reshape(E // SEG, SEG)
    dst = edge_index[0].astype(jnp.int32).reshape(E // SEG, SEG)
    pad_n = ((0, NPAD - N), (0, 0))
    xq = [xs_pad] + [jnp.pad(x_vector[:, q, :], pad_n) for q in range(3)]

    o0, o1, o2, o3 = _sc_message(sout, xvec, p, src, dst, xq)
    new_scalar = o0[:N]
    new_vector = jnp.stack([o1[:N], o2[:N], o3[:N]], axis=1)
    return new_scalar, new_vector
